# baseline (device time: 236879 ns/iter reference)
import functools

import jax
import jax.numpy as jnp
from jax import lax
from jax.experimental import pallas as pl
from jax.experimental.pallas import tpu as pltpu

N_DEV = 8
B, SQ, D = 4, 256, 1024
HQ, HKV, DH = 8, 2, 128
SKV_LOCAL = 1024
SCALE = 0.08838834764831843
R = B * SQ


def _partial_body(x_ref, wq_ref, k_ref, v_ref, o_ref, m_ref, l_ref):
    X = x_ref[...].reshape(R, D).astype(jnp.bfloat16)
    Wq = wq_ref[...].astype(jnp.bfloat16)
    Q = jnp.dot(X, Wq, preferred_element_type=jnp.float32)
    Q = (Q * SCALE).astype(jnp.bfloat16).reshape(B, SQ, HQ, DH)
    for b in range(B):
        for g in range(HKV):
            Qg = Q[b, :, 4 * g : 4 * g + 4, :].reshape(SQ * 4, DH)
            Kg = k_ref[b, :, g, :].astype(jnp.bfloat16)
            Vg = v_ref[b, :, g, :].astype(jnp.bfloat16)
            s = lax.dot_general(
                Qg, Kg, (((1,), (1,)), ((), ())),
                preferred_element_type=jnp.float32,
            )
            mx = jnp.max(s, axis=1)
            p = jnp.exp(s - mx[:, None])
            ls = jnp.sum(p, axis=1)
            o = jnp.dot(
                p.astype(jnp.bfloat16), Vg, preferred_element_type=jnp.float32
            )
            o_ref[b, g] = o.astype(jnp.bfloat16)
            m_ref[2 * b + g, :] = mx
            l_ref[2 * b + g, :] = ls


def _ring_body(
    op_ref, m_ref, l_ref, wo_ref, out_ref,
    comm_ref, mlc_ref, mlown_ref,
    acc_o_ref, acc_m_ref, acc_l_ref,
    o_send, o_recv, ml_send, ml_recv,
):
    me = lax.axis_index("i")
    left = lax.rem(me + N_DEV - 1, N_DEV)
    right = lax.rem(me + 1, N_DEV)

    barrier_sem = pltpu.get_barrier_semaphore()
    for nbr in [left, right]:
        pl.semaphore_signal(
            barrier_sem, inc=1,
            device_id=(nbr,), device_id_type=pl.DeviceIdType.MESH,
        )
    pl.semaphore_wait(barrier_sem, 2)

    mlown_ref[0] = m_ref[...]
    mlown_ref[1] = l_ref[...]
    acc_o_ref[...] = op_ref[...].astype(jnp.float32)
    acc_m_ref[...] = m_ref[...]
    acc_l_ref[...] = l_ref[...]

    for h in range(N_DEV - 1):
        src_o = op_ref if h == 0 else comm_ref.at[h - 1]
        src_ml = mlown_ref if h == 0 else mlc_ref.at[h - 1]
        rdma_o = pltpu.make_async_remote_copy(
            src_ref=src_o, dst_ref=comm_ref.at[h],
            send_sem=o_send.at[h], recv_sem=o_recv.at[h],
            device_id=(right,), device_id_type=pl.DeviceIdType.MESH,
        )
        rdma_ml = pltpu.make_async_remote_copy(
            src_ref=src_ml, dst_ref=mlc_ref.at[h],
            send_sem=ml_send.at[h], recv_sem=ml_recv.at[h],
            device_id=(right,), device_id_type=pl.DeviceIdType.MESH,
        )
        rdma_o.start()
        rdma_ml.start()
        rdma_o.wait()
        rdma_ml.wait()

        m_in = mlc_ref[h, 0]
        l_in = mlc_ref[h, 1]
        o_in = comm_ref[h].astype(jnp.float32)
        m_old = acc_m_ref[...]
        m_new = jnp.maximum(m_old, m_in)
        a_old = jnp.exp(m_old - m_new)
        a_in = jnp.exp(m_in - m_new)
        acc_m_ref[...] = m_new
        acc_l_ref[...] = acc_l_ref[...] * a_old + l_in * a_in
        acc_o_ref[...] = (
            acc_o_ref[...] * a_old.reshape(B, HKV, SQ * 4, 1)
            + o_in * a_in.reshape(B, HKV, SQ * 4, 1)
        )

    o = acc_o_ref[...] / acc_l_ref[...].reshape(B, HKV, SQ * 4, 1)
    o = o.astype(jnp.bfloat16)
    Wo = wo_ref[...].astype(jnp.bfloat16)
    for b in range(B):
        ob = jnp.concatenate(
            [o[b, g].reshape(SQ, 4 * DH) for g in range(HKV)], axis=1
        )
        out_ref[b] = jnp.dot(ob, Wo, preferred_element_type=jnp.float32)

    @functools.partial(
        pl.run_scoped, second_barrier=pltpu.SemaphoreType.REGULAR
    )
    def _(second_barrier):
        for nbr in [left, right]:
            pl.semaphore_signal(
                second_barrier, inc=1,
                device_id=(nbr,), device_id_type=pl.DeviceIdType.MESH,
            )
        pl.semaphore_wait(second_barrier, 2)


def kernel(x, Wq, Wo, K_ext, V_ext):
    op, m, l = pl.pallas_call(
        _partial_body,
        out_shape=(
            jax.ShapeDtypeStruct((B, HKV, SQ * 4, DH), jnp.bfloat16),
            jax.ShapeDtypeStruct((B * HKV, R), jnp.float32),
            jax.ShapeDtypeStruct((B * HKV, R), jnp.float32),
        ),
        in_specs=[pl.BlockSpec(memory_space=pltpu.VMEM)] * 4,
        out_specs=(
            pl.BlockSpec(memory_space=pltpu.VMEM),
            pl.BlockSpec(memory_space=pltpu.VMEM),
            pl.BlockSpec(memory_space=pltpu.VMEM),
        ),
    )(x, Wq, K_ext, V_ext)

    out = pl.pallas_call(
        _ring_body,
        out_shape=jax.ShapeDtypeStruct((B, SQ, D), jnp.float32),
        in_specs=[pl.BlockSpec(memory_space=pltpu.VMEM)] * 4,
        out_specs=pl.BlockSpec(memory_space=pltpu.VMEM),
        scratch_shapes=[
            pltpu.VMEM((N_DEV - 1, B, HKV, SQ * 4, DH), jnp.bfloat16),
            pltpu.VMEM((N_DEV - 1, 2, B * HKV, R), jnp.float32),
            pltpu.VMEM((2, B * HKV, R), jnp.float32),
            pltpu.VMEM((B, HKV, SQ * 4, DH), jnp.float32),
            pltpu.VMEM((B * HKV, R), jnp.float32),
            pltpu.VMEM((B * HKV, R), jnp.float32),
            pltpu.SemaphoreType.DMA((N_DEV - 1,)),
            pltpu.SemaphoreType.DMA((N_DEV - 1,)),
            pltpu.SemaphoreType.DMA((N_DEV - 1,)),
            pltpu.SemaphoreType.DMA((N_DEV - 1,)),
        ],
        compiler_params=pltpu.CompilerParams(collective_id=0),
    )(op, m, l, Wo)
    return out


# device time: 105814 ns/iter; 2.2386x vs baseline; 2.2386x over previous
import functools

import jax
import jax.numpy as jnp
from jax import lax
from jax.experimental import pallas as pl
from jax.experimental.pallas import tpu as pltpu

N_DEV = 8
B, SQ, D = 4, 256, 1024
HQ, HKV, DH = 8, 2, 128
SCALE = 0.08838834764831843
R = B * SQ
BLK = 1024 // N_DEV
SBLK = SQ // N_DEV


def _partial_body(x_ref, wq_ref, k_ref, v_ref, o_ref, ml_ref):
    X = x_ref[...].reshape(R, D).astype(jnp.bfloat16)
    Wq = wq_ref[...].astype(jnp.bfloat16)
    Q = jnp.dot(X, Wq, preferred_element_type=jnp.float32)
    Q = (Q * SCALE).astype(jnp.bfloat16).reshape(B, SQ, HQ, DH)
    for b in range(B):
        for g in range(HKV):
            Qg = Q[b, :, 4 * g : 4 * g + 4, :].reshape(SQ * 4, DH)
            Kg = k_ref[b, :, g, :].astype(jnp.bfloat16)
            Vg = v_ref[b, :, g, :].astype(jnp.bfloat16)
            s = lax.dot_general(
                Qg, Kg, (((1,), (1,)), ((), ())),
                preferred_element_type=jnp.float32,
            )
            mx = jnp.max(s, axis=1)
            p = jnp.exp(s - mx[:, None])
            ls = jnp.sum(p, axis=1)
            o = jnp.dot(
                p.astype(jnp.bfloat16), Vg, preferred_element_type=jnp.float32
            ).astype(jnp.bfloat16)
            for r in range(N_DEV):
                o_ref[r, b, g] = o[r * BLK : (r + 1) * BLK]
                ml_ref[r, 0, 2 * b + g, :] = mx[r * BLK : (r + 1) * BLK]
                ml_ref[r, 1, 2 * b + g, :] = ls[r * BLK : (r + 1) * BLK]


def _ring_body(
    op_ref, ml_ref, wo_ref, out_ref,
    rs_o, rs_ml, cb_o, cb_ml, yown, ag,
    rso_send, rso_recv, rsml_send, rsml_recv, ag_send, ag_recv,
):
    me = lax.axis_index("i")
    left = lax.rem(me + N_DEV - 1, N_DEV)
    right = lax.rem(me + 1, N_DEV)

    barrier_sem = pltpu.get_barrier_semaphore()
    for nbr in [left, right]:
        pl.semaphore_signal(
            barrier_sem, inc=1,
            device_id=(nbr,), device_id_type=pl.DeviceIdType.MESH,
        )
    pl.semaphore_wait(barrier_sem, 2)

    Wo = wo_ref[...].astype(jnp.bfloat16)

    for t in range(N_DEV - 1):
        c_send = lax.rem(me - t + N_DEV, N_DEV)
        src_o = op_ref.at[c_send] if t == 0 else cb_o.at[t - 1]
        src_ml = ml_ref.at[c_send] if t == 0 else cb_ml.at[t - 1]
        rdma_o = pltpu.make_async_remote_copy(
            src_ref=src_o, dst_ref=rs_o.at[t],
            send_sem=rso_send.at[t], recv_sem=rso_recv.at[t],
            device_id=(right,), device_id_type=pl.DeviceIdType.MESH,
        )
        rdma_ml = pltpu.make_async_remote_copy(
            src_ref=src_ml, dst_ref=rs_ml.at[t],
            send_sem=rsml_send.at[t], recv_sem=rsml_recv.at[t],
            device_id=(right,), device_id_type=pl.DeviceIdType.MESH,
        )
        rdma_o.start()
        rdma_ml.start()
        rdma_o.wait()
        rdma_ml.wait()

        c = lax.rem(me - t - 1 + N_DEV, N_DEV)
        my_o = op_ref[c].astype(jnp.float32)
        my_m = ml_ref[c, 0]
        my_l = ml_ref[c, 1]
        in_o = rs_o[t].astype(jnp.float32)
        in_m = rs_ml[t, 0]
        in_l = rs_ml[t, 1]
        m_n = jnp.maximum(my_m, in_m)
        a_my = jnp.exp(my_m - m_n)
        a_in = jnp.exp(in_m - m_n)
        l_n = my_l * a_my + in_l * a_in
        o_n = (
            my_o * a_my.reshape(B, HKV, BLK, 1)
            + in_o * a_in.reshape(B, HKV, BLK, 1)
        )
        if t < N_DEV - 2:
            cb_o[t] = o_n.astype(jnp.bfloat16)
            cb_ml[t, 0] = m_n
            cb_ml[t, 1] = l_n
        else:
            o_f = (o_n / l_n.reshape(B, HKV, BLK, 1)).astype(jnp.bfloat16)
            mat = jnp.stack(
                [
                    jnp.concatenate(
                        [o_f[b, g].reshape(SBLK, 4 * DH) for g in range(HKV)],
                        axis=1,
                    )
                    for b in range(B)
                ]
            ).reshape(B * SBLK, D)
            y = jnp.dot(mat, Wo, preferred_element_type=jnp.float32)
            cm = lax.rem(me + 1, N_DEV)
            out_ref[:, pl.ds(cm * SBLK, SBLK), :] = y.reshape(B, SBLK, D)
            yown[...] = y.astype(jnp.bfloat16)

    for h in range(N_DEV - 1):
        src = yown if h == 0 else ag.at[h - 1]
        rdma = pltpu.make_async_remote_copy(
            src_ref=src, dst_ref=ag.at[h],
            send_sem=ag_send.at[h], recv_sem=ag_recv.at[h],
            device_id=(right,), device_id_type=pl.DeviceIdType.MESH,
        )
        rdma.start()
        rdma.wait()
        c = lax.rem(me - h + N_DEV, N_DEV)
        y_in = ag[h].astype(jnp.float32).reshape(B, SBLK, D)
        out_ref[:, pl.ds(c * SBLK, SBLK), :] = y_in

    @functools.partial(
        pl.run_scoped, second_barrier=pltpu.SemaphoreType.REGULAR
    )
    def _(second_barrier):
        for nbr in [left, right]:
            pl.semaphore_signal(
                second_barrier, inc=1,
                device_id=(nbr,), device_id_type=pl.DeviceIdType.MESH,
            )
        pl.semaphore_wait(second_barrier, 2)


def kernel(x, Wq, Wo, K_ext, V_ext):
    op, ml = pl.pallas_call(
        _partial_body,
        out_shape=(
            jax.ShapeDtypeStruct((N_DEV, B, HKV, BLK, DH), jnp.bfloat16),
            jax.ShapeDtypeStruct((N_DEV, 2, B * HKV, BLK), jnp.float32),
        ),
        in_specs=[pl.BlockSpec(memory_space=pltpu.VMEM)] * 4,
        out_specs=(
            pl.BlockSpec(memory_space=pltpu.VMEM),
            pl.BlockSpec(memory_space=pltpu.VMEM),
        ),
    )(x, Wq, K_ext, V_ext)

    out = pl.pallas_call(
        _ring_body,
        out_shape=jax.ShapeDtypeStruct((B, SQ, D), jnp.float32),
        in_specs=[pl.BlockSpec(memory_space=pltpu.VMEM)] * 3,
        out_specs=pl.BlockSpec(memory_space=pltpu.VMEM),
        scratch_shapes=[
            pltpu.VMEM((N_DEV - 1, B, HKV, BLK, DH), jnp.bfloat16),
            pltpu.VMEM((N_DEV - 1, 2, B * HKV, BLK), jnp.float32),
            pltpu.VMEM((N_DEV - 1, B, HKV, BLK, DH), jnp.bfloat16),
            pltpu.VMEM((N_DEV - 1, 2, B * HKV, BLK), jnp.float32),
            pltpu.VMEM((B * SBLK, D), jnp.bfloat16),
            pltpu.VMEM((N_DEV - 1, B * SBLK, D), jnp.bfloat16),
            pltpu.SemaphoreType.DMA((N_DEV - 1,)),
            pltpu.SemaphoreType.DMA((N_DEV - 1,)),
            pltpu.SemaphoreType.DMA((N_DEV - 1,)),
            pltpu.SemaphoreType.DMA((N_DEV - 1,)),
            pltpu.SemaphoreType.DMA((N_DEV - 1,)),
            pltpu.SemaphoreType.DMA((N_DEV - 1,)),
        ],
        compiler_params=pltpu.CompilerParams(collective_id=0),
    )(op, ml, Wo)
    return out


# device time: 91781 ns/iter; 2.5809x vs baseline; 1.1529x over previous
import functools

import jax
import jax.numpy as jnp
from jax import lax
from jax.experimental import pallas as pl
from jax.experimental.pallas import tpu as pltpu

N_DEV = 8
B, SQ, D = 4, 256, 1024
HQ, HKV, DH = 8, 2, 128
SCALE = 0.08838834764831843
R = B * SQ
BLK = 1024 // N_DEV
SBLK = SQ // N_DEV


def _partial_body(x_ref, wq_ref, k_ref, v_ref, o_ref, ml_ref):
    X = x_ref[...].reshape(R, D).astype(jnp.bfloat16)
    Wq = wq_ref[...].astype(jnp.bfloat16)
    Q = jnp.dot(X, Wq, preferred_element_type=jnp.float32)
    Q = (Q * SCALE).astype(jnp.bfloat16).reshape(B, SQ, HQ, DH)
    for b in range(B):
        for g in range(HKV):
            Qg = Q[b, :, 4 * g : 4 * g + 4, :].reshape(SQ * 4, DH)
            Kg = k_ref[b, :, g, :].astype(jnp.bfloat16)
            Vg = v_ref[b, :, g, :].astype(jnp.bfloat16)
            s = lax.dot_general(
                Qg, Kg, (((1,), (1,)), ((), ())),
                preferred_element_type=jnp.float32,
            )
            mx = jnp.max(s, axis=1)
            p = jnp.exp(s - mx[:, None])
            ls = jnp.sum(p, axis=1)
            o = jnp.dot(
                p.astype(jnp.bfloat16), Vg, preferred_element_type=jnp.float32
            ).astype(jnp.bfloat16)
            for r in range(N_DEV):
                o_ref[r, b, g] = o[r * BLK : (r + 1) * BLK]
                ml_ref[r, 0, 2 * b + g, :] = mx[r * BLK : (r + 1) * BLK]
                ml_ref[r, 1, 2 * b + g, :] = ls[r * BLK : (r + 1) * BLK]


def _combine(my_o, my_m, my_l, in_o, in_m, in_l, nblk):
    m_n = jnp.maximum(my_m, in_m)
    a_my = jnp.exp(my_m - m_n)
    a_in = jnp.exp(in_m - m_n)
    l_n = my_l * a_my + in_l * a_in
    o_n = (
        my_o * a_my.reshape(nblk, B, HKV, BLK, 1)
        + in_o * a_in.reshape(nblk, B, HKV, BLK, 1)
    )
    return o_n, m_n, l_n


def _ring_body(
    op_ref, ml_ref, wo_ref, out_ref,
    ro0, ro1, ro2, rml0, rml1, rml2,
    c0o, c0ml, c1o, c1ml, yall,
    o_send, o_recv, ml_send, ml_recv, ag_send, ag_recv,
):
    me = lax.axis_index("i")
    partners = [
        lax.bitwise_xor(me, 4),
        lax.bitwise_xor(me, 2),
        lax.bitwise_xor(me, 1),
    ]

    barrier_sem = pltpu.get_barrier_semaphore()
    for p in partners:
        pl.semaphore_signal(
            barrier_sem, inc=1,
            device_id=(p,), device_id_type=pl.DeviceIdType.MESH,
        )
    pl.semaphore_wait(barrier_sem, 3)

    Wo = wo_ref[...].astype(jnp.bfloat16)

    kb0 = lax.bitwise_and(me, 4)
    sb0 = lax.bitwise_xor(kb0, 4)
    r_o = pltpu.make_async_remote_copy(
        src_ref=op_ref.at[pl.ds(sb0, 4)], dst_ref=ro0,
        send_sem=o_send.at[0], recv_sem=o_recv.at[0],
        device_id=(partners[0],), device_id_type=pl.DeviceIdType.MESH,
    )
    r_ml = pltpu.make_async_remote_copy(
        src_ref=ml_ref.at[pl.ds(sb0, 4)], dst_ref=rml0,
        send_sem=ml_send.at[0], recv_sem=ml_recv.at[0],
        device_id=(partners[0],), device_id_type=pl.DeviceIdType.MESH,
    )
    r_o.start()
    r_ml.start()
    r_o.wait()
    r_ml.wait()
    myml = ml_ref[pl.ds(kb0, 4)]
    o_n, m_n, l_n = _combine(
        op_ref[pl.ds(kb0, 4)].astype(jnp.float32), myml[:, 0], myml[:, 1],
        ro0[...].astype(jnp.float32), rml0[:, 0], rml0[:, 1], 4,
    )
    c0o[...] = o_n.astype(jnp.bfloat16)
    c0ml[...] = jnp.stack([m_n, l_n], axis=1)

    off_k1 = lax.bitwise_and(me, 2)
    off_s1 = lax.bitwise_xor(off_k1, 2)
    r_o = pltpu.make_async_remote_copy(
        src_ref=c0o.at[pl.ds(off_s1, 2)], dst_ref=ro1,
        send_sem=o_send.at[1], recv_sem=o_recv.at[1],
        device_id=(partners[1],), device_id_type=pl.DeviceIdType.MESH,
    )
    r_ml = pltpu.make_async_remote_copy(
        src_ref=c0ml.at[pl.ds(off_s1, 2)], dst_ref=rml1,
        send_sem=ml_send.at[1], recv_sem=ml_recv.at[1],
        device_id=(partners[1],), device_id_type=pl.DeviceIdType.MESH,
    )
    r_o.start()
    r_ml.start()
    r_o.wait()
    r_ml.wait()
    myml = c0ml[pl.ds(off_k1, 2)]
    o_n, m_n, l_n = _combine(
        c0o[pl.ds(off_k1, 2)].astype(jnp.float32), myml[:, 0], myml[:, 1],
        ro1[...].astype(jnp.float32), rml1[:, 0], rml1[:, 1], 2,
    )
    c1o[...] = o_n.astype(jnp.bfloat16)
    c1ml[...] = jnp.stack([m_n, l_n], axis=1)

    off_k2 = lax.bitwise_and(me, 1)
    off_s2 = lax.bitwise_xor(off_k2, 1)
    r_o = pltpu.make_async_remote_copy(
        src_ref=c1o.at[pl.ds(off_s2, 1)], dst_ref=ro2,
        send_sem=o_send.at[2], recv_sem=o_recv.at[2],
        device_id=(partners[2],), device_id_type=pl.DeviceIdType.MESH,
    )
    r_ml = pltpu.make_async_remote_copy(
        src_ref=c1ml.at[pl.ds(off_s2, 1)], dst_ref=rml2,
        send_sem=ml_send.at[2], recv_sem=ml_recv.at[2],
        device_id=(partners[2],), device_id_type=pl.DeviceIdType.MESH,
    )
    r_o.start()
    r_ml.start()
    r_o.wait()
    r_ml.wait()
    myml = c1ml[pl.ds(off_k2, 1)]
    o_n, m_n, l_n = _combine(
        c1o[pl.ds(off_k2, 1)].astype(jnp.float32), myml[:, 0], myml[:, 1],
        ro2[...].astype(jnp.float32), rml2[:, 0], rml2[:, 1], 1,
    )

    o_f = (o_n / l_n.reshape(1, B, HKV, BLK, 1)).astype(jnp.bfloat16)[0]
    mat = jnp.stack(
        [
            jnp.concatenate(
                [o_f[b, g].reshape(SBLK, 4 * DH) for g in range(HKV)],
                axis=1,
            )
            for b in range(B)
        ]
    ).reshape(B * SBLK, D)
    y = jnp.dot(mat, Wo, preferred_element_type=jnp.float32)
    yall[pl.ds(me, 1)] = y.astype(jnp.bfloat16).reshape(1, B * SBLK, D)

    for k, bit in enumerate([1, 2, 4]):
        ab = lax.bitwise_and(me, (~(bit - 1)) & 7)
        partner = lax.bitwise_xor(me, bit)
        rdma = pltpu.make_async_remote_copy(
            src_ref=yall.at[pl.ds(ab, bit)],
            dst_ref=yall.at[pl.ds(ab, bit)],
            send_sem=ag_send.at[k], recv_sem=ag_recv.at[k],
            device_id=(partner,), device_id_type=pl.DeviceIdType.MESH,
        )
        rdma.start()
        rdma.wait()

    for c in range(N_DEV):
        out_ref[:, c * SBLK : (c + 1) * SBLK, :] = (
            yall[c].astype(jnp.float32).reshape(B, SBLK, D)
        )

    @functools.partial(
        pl.run_scoped, second_barrier=pltpu.SemaphoreType.REGULAR
    )
    def _(second_barrier):
        for p in partners:
            pl.semaphore_signal(
                second_barrier, inc=1,
                device_id=(p,), device_id_type=pl.DeviceIdType.MESH,
            )
        pl.semaphore_wait(second_barrier, 3)


def kernel(x, Wq, Wo, K_ext, V_ext):
    op, ml = pl.pallas_call(
        _partial_body,
        out_shape=(
            jax.ShapeDtypeStruct((N_DEV, B, HKV, BLK, DH), jnp.bfloat16),
            jax.ShapeDtypeStruct((N_DEV, 2, B * HKV, BLK), jnp.float32),
        ),
        in_specs=[pl.BlockSpec(memory_space=pltpu.VMEM)] * 4,
        out_specs=(
            pl.BlockSpec(memory_space=pltpu.VMEM),
            pl.BlockSpec(memory_space=pltpu.VMEM),
        ),
    )(x, Wq, K_ext, V_ext)

    out = pl.pallas_call(
        _ring_body,
        out_shape=jax.ShapeDtypeStruct((B, SQ, D), jnp.float32),
        in_specs=[pl.BlockSpec(memory_space=pltpu.VMEM)] * 3,
        out_specs=pl.BlockSpec(memory_space=pltpu.VMEM),
        scratch_shapes=[
            pltpu.VMEM((4, B, HKV, BLK, DH), jnp.bfloat16),
            pltpu.VMEM((2, B, HKV, BLK, DH), jnp.bfloat16),
            pltpu.VMEM((1, B, HKV, BLK, DH), jnp.bfloat16),
            pltpu.VMEM((4, 2, B * HKV, BLK), jnp.float32),
            pltpu.VMEM((2, 2, B * HKV, BLK), jnp.float32),
            pltpu.VMEM((1, 2, B * HKV, BLK), jnp.float32),
            pltpu.VMEM((4, B, HKV, BLK, DH), jnp.bfloat16),
            pltpu.VMEM((4, 2, B * HKV, BLK), jnp.float32),
            pltpu.VMEM((2, B, HKV, BLK, DH), jnp.bfloat16),
            pltpu.VMEM((2, 2, B * HKV, BLK), jnp.float32),
            pltpu.VMEM((N_DEV, B * SBLK, D), jnp.bfloat16),
            pltpu.SemaphoreType.DMA((3,)),
            pltpu.SemaphoreType.DMA((3,)),
            pltpu.SemaphoreType.DMA((3,)),
            pltpu.SemaphoreType.DMA((3,)),
            pltpu.SemaphoreType.DMA((3,)),
            pltpu.SemaphoreType.DMA((3,)),
        ],
        compiler_params=pltpu.CompilerParams(collective_id=0),
    )(op, ml, Wo)
    return out


# device time: 89499 ns/iter; 2.6467x vs baseline; 1.0255x over previous
import functools

import jax
import jax.numpy as jnp
from jax import lax
from jax.experimental import pallas as pl
from jax.experimental.pallas import tpu as pltpu

N_DEV = 8
B, SQ, D = 4, 256, 1024
HQ, HKV, DH = 8, 2, 128
SCALE = 0.08838834764831843
R = B * SQ
BLK = 1024 // N_DEV
SBLK = SQ // N_DEV
HALF = SQ // 2


def _combine(my_o, my_m, my_l, in_o, in_m, in_l, nblk):
    m_n = jnp.maximum(my_m, in_m)
    a_my = jnp.exp(my_m - m_n)
    a_in = jnp.exp(in_m - m_n)
    l_n = my_l * a_my + in_l * a_in
    o_n = (
        my_o * a_my.reshape(nblk, B, HKV, BLK, 1)
        + in_o * a_in.reshape(nblk, B, HKV, BLK, 1)
    )
    return o_n, m_n, l_n


def _fused_body(
    x_ref, wq_ref, wo_ref, k_ref, v_ref, out_ref,
    q_ref, op_ref, ml_ref,
    ro0, ro1, ro2, rml0, rml1, rml2,
    c0o, c0ml, c1o, c1ml, yall,
    o_send, o_recv, ml_send, ml_recv, ag_send, ag_recv,
):
    me = lax.axis_index("i")
    partners = [
        lax.bitwise_xor(me, 4),
        lax.bitwise_xor(me, 2),
        lax.bitwise_xor(me, 1),
    ]

    barrier_sem = pltpu.get_barrier_semaphore()
    for p in partners:
        pl.semaphore_signal(
            barrier_sem, inc=1,
            device_id=(p,), device_id_type=pl.DeviceIdType.MESH,
        )
    pl.semaphore_wait(barrier_sem, 3)

    X = x_ref[...].reshape(R, D).astype(jnp.bfloat16)
    Wq = wq_ref[...].astype(jnp.bfloat16)
    Q = jnp.dot(X, Wq, preferred_element_type=jnp.float32)
    q_ref[...] = (Q * SCALE).astype(jnp.bfloat16).reshape(B, SQ, HQ, DH)

    def compute_half(base):
        for b in range(B):
            Qb = q_ref[b, pl.ds(base * SBLK, HALF)]
            for g in range(HKV):
                Qg = Qb[:, 4 * g : 4 * g + 4, :].reshape(HALF * 4, DH)
                Kg = k_ref[b, :, g, :].astype(jnp.bfloat16)
                Vg = v_ref[b, :, g, :].astype(jnp.bfloat16)
                s = lax.dot_general(
                    Qg, Kg, (((1,), (1,)), ((), ())),
                    preferred_element_type=jnp.float32,
                )
                mx = jnp.max(s, axis=1)
                p = jnp.exp(s - mx[:, None])
                ls = jnp.sum(p, axis=1)
                o = jnp.dot(
                    p.astype(jnp.bfloat16), Vg,
                    preferred_element_type=jnp.float32,
                ).astype(jnp.bfloat16)
                op_ref[pl.ds(base, 4), b, g] = o.reshape(4, BLK, DH)
                ml_ref[pl.ds(base, 4), 0, 2 * b + g] = mx.reshape(4, BLK)
                ml_ref[pl.ds(base, 4), 1, 2 * b + g] = ls.reshape(4, BLK)

    kb0 = lax.bitwise_and(me, 4)
    sb0 = lax.bitwise_xor(kb0, 4)
    compute_half(sb0)
    r_o = pltpu.make_async_remote_copy(
        src_ref=op_ref.at[pl.ds(sb0, 4)], dst_ref=ro0,
        send_sem=o_send.at[0], recv_sem=o_recv.at[0],
        device_id=(partners[0],), device_id_type=pl.DeviceIdType.MESH,
    )
    r_ml = pltpu.make_async_remote_copy(
        src_ref=ml_ref.at[pl.ds(sb0, 4)], dst_ref=rml0,
        send_sem=ml_send.at[0], recv_sem=ml_recv.at[0],
        device_id=(partners[0],), device_id_type=pl.DeviceIdType.MESH,
    )
    r_o.start()
    r_ml.start()
    compute_half(kb0)
    r_o.wait()
    r_ml.wait()
    myml = ml_ref[pl.ds(kb0, 4)]
    o_n, m_n, l_n = _combine(
        op_ref[pl.ds(kb0, 4)].astype(jnp.float32), myml[:, 0], myml[:, 1],
        ro0[...].astype(jnp.float32), rml0[:, 0], rml0[:, 1], 4,
    )
    c0o[...] = o_n.astype(jnp.bfloat16)
    c0ml[...] = jnp.stack([m_n, l_n], axis=1)

    off_k1 = lax.bitwise_and(me, 2)
    off_s1 = lax.bitwise_xor(off_k1, 2)
    r_o = pltpu.make_async_remote_copy(
        src_ref=c0o.at[pl.ds(off_s1, 2)], dst_ref=ro1,
        send_sem=o_send.at[1], recv_sem=o_recv.at[1],
        device_id=(partners[1],), device_id_type=pl.DeviceIdType.MESH,
    )
    r_ml = pltpu.make_async_remote_copy(
        src_ref=c0ml.at[pl.ds(off_s1, 2)], dst_ref=rml1,
        send_sem=ml_send.at[1], recv_sem=ml_recv.at[1],
        device_id=(partners[1],), device_id_type=pl.DeviceIdType.MESH,
    )
    r_o.start()
    r_ml.start()
    r_o.wait()
    r_ml.wait()
    myml = c0ml[pl.ds(off_k1, 2)]
    o_n, m_n, l_n = _combine(
        c0o[pl.ds(off_k1, 2)].astype(jnp.float32), myml[:, 0], myml[:, 1],
        ro1[...].astype(jnp.float32), rml1[:, 0], rml1[:, 1], 2,
    )
    c1o[...] = o_n.astype(jnp.bfloat16)
    c1ml[...] = jnp.stack([m_n, l_n], axis=1)

    off_k2 = lax.bitwise_and(me, 1)
    off_s2 = lax.bitwise_xor(off_k2, 1)
    r_o = pltpu.make_async_remote_copy(
        src_ref=c1o.at[pl.ds(off_s2, 1)], dst_ref=ro2,
        send_sem=o_send.at[2], recv_sem=o_recv.at[2],
        device_id=(partners[2],), device_id_type=pl.DeviceIdType.MESH,
    )
    r_ml = pltpu.make_async_remote_copy(
        src_ref=c1ml.at[pl.ds(off_s2, 1)], dst_ref=rml2,
        send_sem=ml_send.at[2], recv_sem=ml_recv.at[2],
        device_id=(partners[2],), device_id_type=pl.DeviceIdType.MESH,
    )
    r_o.start()
    r_ml.start()
    r_o.wait()
    r_ml.wait()
    myml = c1ml[pl.ds(off_k2, 1)]
    o_n, m_n, l_n = _combine(
        c1o[pl.ds(off_k2, 1)].astype(jnp.float32), myml[:, 0], myml[:, 1],
        ro2[...].astype(jnp.float32), rml2[:, 0], rml2[:, 1], 1,
    )

    Wo = wo_ref[...].astype(jnp.bfloat16)
    o_f = (o_n / l_n.reshape(1, B, HKV, BLK, 1)).astype(jnp.bfloat16)[0]
    mat = jnp.stack(
        [
            jnp.concatenate(
                [o_f[b, g].reshape(SBLK, 4 * DH) for g in range(HKV)],
                axis=1,
            )
            for b in range(B)
        ]
    ).reshape(B * SBLK, D)
    y = jnp.dot(mat, Wo, preferred_element_type=jnp.float32)
    yall[pl.ds(me, 1)] = y.astype(jnp.bfloat16).reshape(1, B * SBLK, D)

    rdmas = []
    for k, bit in enumerate([1, 2, 4]):
        ab = lax.bitwise_and(me, (~(bit - 1)) & 7)
        partner = lax.bitwise_xor(me, bit)
        rdma = pltpu.make_async_remote_copy(
            src_ref=yall.at[pl.ds(ab, bit)],
            dst_ref=yall.at[pl.ds(ab, bit)],
            send_sem=ag_send.at[k], recv_sem=ag_recv.at[k],
            device_id=(partner,), device_id_type=pl.DeviceIdType.MESH,
        )
        rdma.start()
        rdma.wait()
        rdmas.append(rdma)
        rb = lax.bitwise_xor(ab, bit)
        chunk = yall[pl.ds(rb, bit)].astype(jnp.float32)
        chunk = chunk.reshape(bit, B, SBLK, D).transpose(1, 0, 2, 3)
        out_ref[:, pl.ds(rb * SBLK, bit * SBLK), :] = chunk.reshape(
            B, bit * SBLK, D
        )

    out_ref[:, pl.ds(me * SBLK, SBLK), :] = y.reshape(B, SBLK, D)

    @functools.partial(
        pl.run_scoped, second_barrier=pltpu.SemaphoreType.REGULAR
    )
    def _(second_barrier):
        for p in partners:
            pl.semaphore_signal(
                second_barrier, inc=1,
                device_id=(p,), device_id_type=pl.DeviceIdType.MESH,
            )
        pl.semaphore_wait(second_barrier, 3)


def kernel(x, Wq, Wo, K_ext, V_ext):
    return pl.pallas_call(
        _fused_body,
        out_shape=jax.ShapeDtypeStruct((B, SQ, D), jnp.float32),
        in_specs=[pl.BlockSpec(memory_space=pltpu.VMEM)] * 5,
        out_specs=pl.BlockSpec(memory_space=pltpu.VMEM),
        scratch_shapes=[
            pltpu.VMEM((B, SQ, HQ, DH), jnp.bfloat16),
            pltpu.VMEM((N_DEV, B, HKV, BLK, DH), jnp.bfloat16),
            pltpu.VMEM((N_DEV, 2, B * HKV, BLK), jnp.float32),
            pltpu.VMEM((4, B, HKV, BLK, DH), jnp.bfloat16),
            pltpu.VMEM((2, B, HKV, BLK, DH), jnp.bfloat16),
            pltpu.VMEM((1, B, HKV, BLK, DH), jnp.bfloat16),
            pltpu.VMEM((4, 2, B * HKV, BLK), jnp.float32),
            pltpu.VMEM((2, 2, B * HKV, BLK), jnp.float32),
            pltpu.VMEM((1, 2, B * HKV, BLK), jnp.float32),
            pltpu.VMEM((4, B, HKV, BLK, DH), jnp.bfloat16),
            pltpu.VMEM((4, 2, B * HKV, BLK), jnp.float32),
            pltpu.VMEM((2, B, HKV, BLK, DH), jnp.bfloat16),
            pltpu.VMEM((2, 2, B * HKV, BLK), jnp.float32),
            pltpu.VMEM((N_DEV, B * SBLK, D), jnp.bfloat16),
            pltpu.SemaphoreType.DMA((3,)),
            pltpu.SemaphoreType.DMA((3,)),
            pltpu.SemaphoreType.DMA((3,)),
            pltpu.SemaphoreType.DMA((3,)),
            pltpu.SemaphoreType.DMA((3,)),
            pltpu.SemaphoreType.DMA((3,)),
        ],
        compiler_params=pltpu.CompilerParams(collective_id=0),
    )(x, Wq, Wo, K_ext, V_ext)
